# baseline (speedup 1.0000x reference)
import jax
import jax.numpy as jnp
from jax.experimental import pallas as pl
from jax.experimental.pallas import tpu as pltpu

_CH = 400_000
_NCH = 5


def _body(x_hbm, out_ref, buf, sems):
    t = pl.program_id(0)

    def _cp(idx, row):
        return pltpu.make_async_copy(
            x_hbm.at[0, pl.ds(1 + row, 1), pl.ds(idx * _CH, _CH)],
            buf.at[idx % 2, pl.ds(row, 1)],
            sems.at[idx % 2, row],
        )

    @pl.when(t == 0)
    def _init():
        out_ref[0, 0] = 0
        _cp(0, 0).start()
        _cp(0, 1).start()

    @pl.when(t + 1 < _NCH)
    def _prefetch():
        _cp(t + 1, 0).start()
        _cp(t + 1, 1).start()

    _cp(t, 0).wait()
    _cp(t, 1).wait()
    nz = jnp.any(buf[t % 2] != 0.0).astype(jnp.int32)
    out_ref[0, 0] = out_ref[0, 0] | nz


def kernel(x):
    xt = jnp.swapaxes(x, 0, 1)
    res = pl.pallas_call(
        _body,
        grid=(_NCH,),
        in_specs=[pl.BlockSpec(memory_space=pl.ANY)],
        out_specs=pl.BlockSpec(memory_space=pltpu.SMEM),
        out_shape=jax.ShapeDtypeStruct((1, 1), jnp.int32),
        scratch_shapes=[
            pltpu.VMEM((2, 2, _CH), jnp.float32),
            pltpu.SemaphoreType.DMA((2, 2)),
        ],
    )(xt)
    return (res[0, 0] != 0).reshape(1)


# speedup vs baseline: 1.0076x; 1.0076x over previous
import jax
import jax.numpy as jnp
from jax.experimental import pallas as pl
from jax.experimental.pallas import tpu as pltpu

_CH = 400_000
_NCH = 5


def _body(x_hbm, out_ref, buf, sems):
    t = pl.program_id(0)

    def _cp(idx):
        return pltpu.make_async_copy(
            x_hbm.at[0, pl.ds(1, 2), pl.ds(idx * _CH, _CH)],
            buf.at[idx % 3],
            sems.at[idx % 3],
        )

    @pl.when(t == 0)
    def _init():
        out_ref[0, 0] = 0
        for k in range(2):
            _cp(k).start()

    @pl.when(t + 2 < _NCH)
    def _prefetch():
        _cp(t + 2).start()

    _cp(t).wait()
    nz = jnp.any(buf[t % 3] != 0.0).astype(jnp.int32)
    out_ref[0, 0] = out_ref[0, 0] | nz


def kernel(x):
    xt = jnp.swapaxes(x, 0, 1)
    res = pl.pallas_call(
        _body,
        grid=(_NCH,),
        in_specs=[pl.BlockSpec(memory_space=pl.ANY)],
        out_specs=pl.BlockSpec(memory_space=pltpu.SMEM),
        out_shape=jax.ShapeDtypeStruct((1, 1), jnp.int32),
        compiler_params=pltpu.CompilerParams(vmem_limit_bytes=100 * 1024 * 1024),
        scratch_shapes=[
            pltpu.VMEM((3, 2, _CH), jnp.float32),
            pltpu.SemaphoreType.DMA((3,)),
        ],
    )(xt)
    return (res[0, 0] != 0).reshape(1)
